# baseline (speedup 1.0000x reference)
import functools

import numpy as np
import jax
import jax.numpy as jnp
from jax import lax
from jax.experimental import pallas as pl
from jax.experimental.pallas import tpu as pltpu
from jax.experimental.pallas import tpu_sc as plsc

_PRIMES = [49999, 49993, 49991, 49957, 49943, 49939, 49937, 49927]
_H = len(_PRIMES)
_HC = 4
_HID = 1024
_DH = 128
_KSZ = 4
_DIL = 3
_EPS_G = float(np.finfo(np.float32).eps)
_EPS_C = 1e-5
_OFFSETS = np.concatenate(
    [[0], np.cumsum(np.asarray(_PRIMES[:-1], dtype=np.int64))]
).astype(np.int32)

_NW = 32
_CHUNK = 128


def _sc_gather(table, idx, n_tokens):
    nw, nchunks, c = idx.shape
    mesh = plsc.VectorSubcoreMesh(core_axis_name="c", subcore_axis_name="s")

    @functools.partial(
        pl.kernel,
        out_type=jax.ShapeDtypeStruct((n_tokens, _H * _DH), jnp.float32),
        mesh=mesh,
        scratch_types=[
            pltpu.VMEM((nchunks, c), jnp.int32),
            pltpu.VMEM((c, _DH), jnp.float32),
            pltpu.VMEM((c, _DH), jnp.float32),
            pltpu.SemaphoreType.DMA,
            pltpu.SemaphoreType.DMA,
        ],
    )
    def gather_kernel(table_hbm, idx_hbm, out_hbm, idx_v, buf0, buf1, sem0, sem1):
        wid = lax.axis_index("s") * 2 + lax.axis_index("c")
        pltpu.sync_copy(idx_hbm.at[wid], idx_v)

        bufs = (buf0, buf1)
        sems = (sem0, sem1)
        pltpu.async_copy(table_hbm.at[idx_v.at[0]], buf0, sem0)

        def step(j, _):
            g = wid * nchunks + j
            tb = g // _H
            h = g % _H

            def even_odd(parity):
                buf, sem = bufs[parity], sems[parity]
                nbuf, nsem = bufs[1 - parity], sems[1 - parity]

                @pl.when(j + 1 < nchunks)
                def _():
                    pltpu.async_copy(table_hbm.at[idx_v.at[j + 1]], nbuf, nsem)

                pltpu.make_async_copy(table_hbm.at[idx_v.at[j]], buf, sem).wait()
                pltpu.sync_copy(
                    buf,
                    out_hbm.at[pl.ds(tb * c, c), pl.ds(h * _DH, _DH)],
                )

            @pl.when(j % 2 == 0)
            def _():
                even_odd(0)

            @pl.when(j % 2 == 1)
            def _():
                even_odd(1)

            return 0

        lax.fori_loop(0, nchunks, step, 0)

    return gather_kernel(table, idx)


def _dense(emb, hid, wcat, wg, norms_w, convw):
    n = emb.shape[0]
    t = 256
    grid = n // t
    cpb = 2048 // t
    halo = 16

    def body(emb_ref, hid_ref, w_ref, wg_ref, nw_ref, cw_ref,
             out_ref, tail_ref):
        i = pl.program_id(0)
        seq_start = (i % cpb) == 0
        eb = emb_ref[...].astype(jnp.bfloat16)
        p = lax.dot_general(eb, w_ref[...],
                            (((1,), (1,)), ((), ())),
                            preferred_element_type=jnp.float32)
        value = p[:, :_HID]
        mv = jnp.mean(value * value, axis=-1, keepdims=True)
        for m in range(_HC):
            k = p[:, _HID * (m + 1):_HID * (m + 2)]
            q = hid_ref[:, m, :]
            rsk = lax.rsqrt(jnp.mean(k * k, axis=-1, keepdims=True) + _EPS_G)
            rsq = lax.rsqrt(jnp.mean(q * q, axis=-1, keepdims=True) + _EPS_G)
            kq = jnp.sum(k * q * wg_ref[m][None, :], axis=-1, keepdims=True)
            g = kq * rsk * rsq * (1.0 / np.sqrt(float(_HID)))
            g = jnp.sqrt(jnp.clip(jnp.abs(g), 1e-6, None)) * jnp.sign(g)
            gate = jax.nn.sigmoid(g)
            gated = gate * value
            rsg = lax.rsqrt(gate * gate * mv + _EPS_C)
            xs = gated * rsg * nw_ref[m][None, :]
            prev_tail = jnp.where(seq_start, 0.0, tail_ref[m])
            xfull = jnp.concatenate([prev_tail, xs], axis=0)
            co = jnp.zeros_like(xs)
            for kk in range(_KSZ):
                off = halo - (_KSZ - 1) * _DIL + _DIL * kk
                co = co + xfull[off:off + t, :] \
                    * cw_ref[kk, _HID * m:_HID * (m + 1)][None, :]
            co = co * jax.nn.sigmoid(co)
            tail_ref[m] = xs[t - halo:, :]
            out_ref[:, m, :] = q + co + gated

    call = pl.pallas_call(
        body,
        grid=(grid,),
        in_specs=[
            pl.BlockSpec((t, _H * _DH), lambda i: (i, 0)),
            pl.BlockSpec((t, _HC, _HID), lambda i: (i, 0, 0)),
            pl.BlockSpec(((1 + _HC) * _HID, _H * _DH), lambda i: (0, 0)),
            pl.BlockSpec((_HC, _HID), lambda i: (0, 0)),
            pl.BlockSpec((_HC, _HID), lambda i: (0, 0)),
            pl.BlockSpec((_KSZ, _HC * _HID), lambda i: (0, 0)),
        ],
        out_specs=pl.BlockSpec((t, _HC, _HID), lambda i: (i, 0, 0)),
        out_shape=jax.ShapeDtypeStruct((n, _HC, _HID), jnp.float32),
        scratch_shapes=[pltpu.VMEM((_HC, 16, _HID), jnp.float32)],
    )
    return call(emb, hid, wcat, wg, norms_w, convw)


def kernel(hash_indices, hidden_states, table, w_v, w_k, g_k, g_h,
           norms_w, conv_w):
    b, s, h = hash_indices.shape
    n = b * s
    idx = hash_indices.astype(jnp.int32) + jnp.asarray(_OFFSETS, jnp.int32)
    idx = (idx.reshape(n // _CHUNK, _CHUNK, _H)
           .transpose(0, 2, 1)
           .reshape(_NW, -1, _CHUNK))
    emb = _sc_gather(table, idx, n)
    hid = hidden_states.reshape(n, _HC, _HID)
    wcat = jnp.concatenate(
        [w_v, w_k.reshape(_HC * _HID, _H * _DH)], axis=0
    ).astype(jnp.bfloat16)
    convw = conv_w.reshape(_HC * _HID, _KSZ).T
    out = _dense(emb, hid, wcat, g_k * g_h, norms_w, convw)
    return out.reshape(b, s, _HC, _HID)


# speedup vs baseline: 1.3901x; 1.3901x over previous
import functools

import numpy as np
import jax
import jax.numpy as jnp
from jax import lax
from jax.experimental import pallas as pl
from jax.experimental.pallas import tpu as pltpu
from jax.experimental.pallas import tpu_sc as plsc

_PRIMES = [49999, 49993, 49991, 49957, 49943, 49939, 49937, 49927]
_H = len(_PRIMES)
_HC = 4
_HID = 1024
_DH = 128
_KSZ = 4
_DIL = 3
_EPS_G = float(np.finfo(np.float32).eps)
_EPS_C = 1e-5
_OFFSETS = np.concatenate(
    [[0], np.cumsum(np.asarray(_PRIMES[:-1], dtype=np.int64))]
).astype(np.int32)

_NW = 32
_CHUNK = 128


def _sc_gather(table, idx, n_tokens):
    nw, nchunks, c = idx.shape
    mesh = plsc.VectorSubcoreMesh(core_axis_name="c", subcore_axis_name="s")

    @functools.partial(
        pl.kernel,
        out_type=jax.ShapeDtypeStruct((n_tokens, _H * _DH), jnp.float32),
        mesh=mesh,
        scratch_types=[
            pltpu.VMEM((nchunks, c), jnp.int32),
            pltpu.VMEM((c, _DH), jnp.float32),
            pltpu.VMEM((c, _DH), jnp.float32),
            pltpu.SemaphoreType.DMA,
            pltpu.SemaphoreType.DMA,
        ],
    )
    def gather_kernel(table_hbm, idx_hbm, out_hbm, idx_v, buf0, buf1, sem0, sem1):
        wid = lax.axis_index("s") * 2 + lax.axis_index("c")
        pltpu.sync_copy(idx_hbm.at[wid], idx_v)

        bufs = (buf0, buf1)
        sems = (sem0, sem1)
        pltpu.async_copy(table_hbm.at[idx_v.at[0]], buf0, sem0)

        def step(j, _):
            g = wid * nchunks + j
            tb = g // _H
            h = g % _H

            def even_odd(parity):
                buf, sem = bufs[parity], sems[parity]
                nbuf, nsem = bufs[1 - parity], sems[1 - parity]

                @pl.when(j + 1 < nchunks)
                def _():
                    pltpu.async_copy(table_hbm.at[idx_v.at[j + 1]], nbuf, nsem)

                pltpu.make_async_copy(table_hbm.at[idx_v.at[j]], buf, sem).wait()
                pltpu.sync_copy(
                    buf,
                    out_hbm.at[pl.ds(tb * c, c), pl.ds(h * _DH, _DH)],
                )

            @pl.when(j % 2 == 0)
            def _():
                even_odd(0)

            @pl.when(j % 2 == 1)
            def _():
                even_odd(1)

            return 0

        lax.fori_loop(0, nchunks, step, 0)

    return gather_kernel(table, idx)


def _dense(emb, hid, wcat, wg, norms_w, convw):
    n = emb.shape[0]
    t = 256
    grid = n // t
    cpb = 2048 // t
    halo = 16

    def body(emb_ref, hid_hbm, w_ref, pmat_ref, wg_ref, nw_ref, cw_ref,
             out_hbm, tail_ref, hid_buf, out_buf, hsem, osem):
        i = pl.program_id(0)
        slot = i % 2
        seq_start = (i % cpb) == 0

        def hid_cp(blk, sl, m):
            return pltpu.make_async_copy(
                hid_hbm.at[pl.ds(blk * t, t), m],
                hid_buf.at[sl, m], hsem.at[sl])

        def out_cp(blk, sl, m):
            return pltpu.make_async_copy(
                out_buf.at[sl, m],
                out_hbm.at[pl.ds(blk * t, t), m], osem.at[sl])

        @pl.when(i == 0)
        def _():
            for m in range(_HC):
                hid_cp(0, 0, m).start()

        @pl.when(i + 1 < grid)
        def _():
            for m in range(_HC):
                hid_cp(i + 1, (i + 1) % 2, m).start()

        for m in range(_HC):
            hid_cp(i, slot, m).wait()

        @pl.when(i >= 2)
        def _():
            for m in range(_HC):
                out_cp(i - 2, slot, m).wait()

        eb = emb_ref[...].astype(jnp.bfloat16)
        p = lax.dot_general(eb, w_ref[...],
                            (((1,), (1,)), ((), ())),
                            preferred_element_type=jnp.float32)
        value = p[:, :_HID]
        mv = jnp.mean(value * value, axis=-1, keepdims=True)
        for m in range(_HC):
            k = p[:, _HID * (m + 1):_HID * (m + 2)]
            q = hid_buf[slot, m]
            rsk = lax.rsqrt(jnp.mean(k * k, axis=-1, keepdims=True) + _EPS_G)
            rsq = lax.rsqrt(jnp.mean(q * q, axis=-1, keepdims=True) + _EPS_G)
            kq = jnp.sum(k * q * wg_ref[m][None, :], axis=-1, keepdims=True)
            g = kq * rsk * rsq * (1.0 / np.sqrt(float(_HID)))
            g = jnp.sqrt(jnp.clip(jnp.abs(g), 1e-6, None)) * jnp.sign(g)
            gate = jax.nn.sigmoid(g)
            gated = gate * value
            rsg = lax.rsqrt(gate * gate * mv + _EPS_C)
            xs = gated * rsg * nw_ref[m][None, :]
            prev_tail = jnp.where(seq_start, 0.0, tail_ref[m])
            xfull = jnp.concatenate(
                [prev_tail, xs], axis=0).astype(jnp.bfloat16)
            sh = lax.dot_general(pmat_ref[...], xfull,
                                 (((1,), (0,)), ((), ())),
                                 preferred_element_type=jnp.float32)
            co = jnp.zeros_like(xs)
            for kk in range(_KSZ):
                co = co + sh[t * kk:t * (kk + 1), :] \
                    * cw_ref[kk, _HID * m:_HID * (m + 1)][None, :]
            co = co * jax.nn.sigmoid(co)
            tail_ref[m] = xs[t - halo:, :]
            out_buf[slot, m] = q + co + gated

        for m in range(_HC):
            out_cp(i, slot, m).start()

        @pl.when(i == grid - 1)
        def _():
            for m in range(_HC):
                out_cp(i - 1, 1 - slot, m).wait()
            for m in range(_HC):
                out_cp(i, slot, m).wait()

    call = pl.pallas_call(
        body,
        grid=(grid,),
        in_specs=[
            pl.BlockSpec((t, _H * _DH), lambda i: (i, 0)),
            pl.BlockSpec(memory_space=pl.ANY),
            pl.BlockSpec(((1 + _HC) * _HID, _H * _DH), lambda i: (0, 0)),
            pl.BlockSpec((_KSZ * t, t + halo), lambda i: (0, 0)),
            pl.BlockSpec((_HC, _HID), lambda i: (0, 0)),
            pl.BlockSpec((_HC, _HID), lambda i: (0, 0)),
            pl.BlockSpec((_KSZ, _HC * _HID), lambda i: (0, 0)),
        ],
        out_specs=pl.BlockSpec(memory_space=pl.ANY),
        out_shape=jax.ShapeDtypeStruct((n, _HC, _HID), jnp.float32),
        scratch_shapes=[
            pltpu.VMEM((_HC, 16, _HID), jnp.float32),
            pltpu.VMEM((2, _HC, t, _HID), jnp.float32),
            pltpu.VMEM((2, _HC, t, _HID), jnp.float32),
            pltpu.SemaphoreType.DMA((2,)),
            pltpu.SemaphoreType.DMA((2,)),
        ],
    )
    pmat = np.zeros((_KSZ * t, t + halo), np.float32)
    for kk in range(_KSZ):
        off = halo - (_KSZ - 1) * _DIL + _DIL * kk
        pmat[t * kk + np.arange(t), np.arange(t) + off] = 1.0
    return call(emb, hid, wcat, jnp.asarray(pmat, jnp.bfloat16), wg,
                norms_w, convw)


def kernel(hash_indices, hidden_states, table, w_v, w_k, g_k, g_h,
           norms_w, conv_w):
    b, s, h = hash_indices.shape
    n = b * s
    idx = hash_indices.astype(jnp.int32) + jnp.asarray(_OFFSETS, jnp.int32)
    idx = (idx.reshape(n // _CHUNK, _CHUNK, _H)
           .transpose(0, 2, 1)
           .reshape(_NW, -1, _CHUNK))
    emb = _sc_gather(table, idx, n)
    hid = hidden_states.reshape(n, _HC, _HID)
    wcat = jnp.concatenate(
        [w_v, w_k.reshape(_HC * _HID, _H * _DH)], axis=0
    ).astype(jnp.bfloat16)
    convw = conv_w.reshape(_HC * _HID, _KSZ).T
    out = _dense(emb, hid, wcat, g_k * g_h, norms_w, convw)
    return out.reshape(b, s, _HC, _HID)
